# Initial kernel scaffold; baseline (speedup 1.0000x reference)
import jax
import jax.numpy as jnp
from jax.experimental import pallas as pl


def kernel(lm_x, vis_x, lm_W1, lm_b1, lm_g1, lm_be1, lm_W2, lm_b2, lm_g2, lm_be2, vis_W1, vis_b1, vis_g1, vis_be1, vis_W2, vis_b2, vis_g2, vis_be2, emb, lm_dec_W, lm_dec_b, vis_dec_W, vis_dec_b):
    raise NotImplementedError("write your pallas kernel here")


# speedup vs baseline: 1.1720x; 1.1720x over previous
import functools

import jax
import jax.numpy as jnp
from jax import lax
from jax.experimental import pallas as pl
from jax.experimental.pallas import tpu as pltpu
from jax.experimental.pallas import tpu_sc as plsc

EPS = 1e-5

_R = 256
_KB = 1024


def _enc_vq_body(x_ref, W1_ref, b1_ref, g1_ref, be1_ref, W2_ref, b2_ref,
                 g2_ref, be2_ref, emb_ref, z_ref, idx_ref, csum_ref):
    x = x_ref[...]
    h = jnp.dot(x, W1_ref[...], preferred_element_type=jnp.float32) + b1_ref[...]
    m = jnp.mean(h, axis=-1, keepdims=True)
    v = jnp.mean((h - m) ** 2, axis=-1, keepdims=True)
    h = (h - m) / jnp.sqrt(v + EPS) * g1_ref[...] + be1_ref[...]
    h = jnp.maximum(h, 0)
    h2 = jnp.dot(h, W2_ref[...], preferred_element_type=jnp.float32) + b2_ref[...]
    m2 = jnp.mean(h2, axis=-1, keepdims=True)
    v2 = jnp.mean((h2 - m2) ** 2, axis=-1, keepdims=True)
    z = (h2 - m2) / jnp.sqrt(v2 + EPS) * g2_ref[...] + be2_ref[...]
    z_ref[...] = z

    sz = jnp.sum(z ** 2, axis=1, keepdims=True)
    K = emb_ref.shape[0]
    run_v = jnp.full((_R,), jnp.inf, jnp.float32)
    run_i = jnp.zeros((_R,), jnp.int32)
    for kb in range(K // _KB):
        e = emb_ref[pl.ds(kb * _KB, _KB), :]
        d = sz - 2.0 * lax.dot_general(
            z, e, (((1,), (1,)), ((), ())),
            preferred_element_type=jnp.float32) + jnp.sum(e ** 2, axis=1)[None, :]
        bv = jnp.min(d, axis=1)
        ii = lax.broadcasted_iota(jnp.int32, d.shape, 1)
        bi = jnp.min(jnp.where(d == bv[:, None], ii, K), axis=1).astype(jnp.int32)
        upd = bv < run_v
        run_i = jnp.where(upd, bi + kb * _KB, run_i)
        run_v = jnp.where(upd, bv, run_v)
    idx_ref[0, 0, :] = run_i

    @pl.when(pl.program_id(0) == 0)
    def _init():
        csum_ref[...] = jnp.zeros((1, 1), jnp.float32)

    csum_ref[...] += jnp.sum(run_v).reshape(1, 1)


def _enc_vq(x, W1, b1, g1, be1, W2, b2, g2, be2, emb):
    N, IN = x.shape
    H = W1.shape[1]
    CD = W2.shape[1]
    K = emb.shape[0]
    grid = N // _R
    z, idx3, csum = pl.pallas_call(
        _enc_vq_body,
        grid=(grid,),
        in_specs=[
            pl.BlockSpec((_R, IN), lambda i: (i, 0)),
            pl.BlockSpec((IN, H), lambda i: (0, 0)),
            pl.BlockSpec((1, H), lambda i: (0, 0)),
            pl.BlockSpec((1, H), lambda i: (0, 0)),
            pl.BlockSpec((1, H), lambda i: (0, 0)),
            pl.BlockSpec((H, CD), lambda i: (0, 0)),
            pl.BlockSpec((1, CD), lambda i: (0, 0)),
            pl.BlockSpec((1, CD), lambda i: (0, 0)),
            pl.BlockSpec((1, CD), lambda i: (0, 0)),
            pl.BlockSpec((K, CD), lambda i: (0, 0)),
        ],
        out_specs=[
            pl.BlockSpec((_R, CD), lambda i: (i, 0)),
            pl.BlockSpec((1, 1, _R), lambda i: (i, 0, 0)),
            pl.BlockSpec((1, 1), lambda i: (0, 0)),
        ],
        out_shape=[
            jax.ShapeDtypeStruct((N, CD), jnp.float32),
            jax.ShapeDtypeStruct((grid, 1, _R), jnp.int32),
            jax.ShapeDtypeStruct((1, 1), jnp.float32),
        ],
        compiler_params=pltpu.CompilerParams(
            dimension_semantics=("arbitrary",)),
    )(x, W1, b1.reshape(1, H), g1.reshape(1, H), be1.reshape(1, H),
      W2, b2.reshape(1, CD), g2.reshape(1, CD), be2.reshape(1, CD), emb)
    return z, idx3.reshape(N), csum


def _recon_body(z_ref, q_ref, W_ref, b_ref, qst_ref, rec_ref):
    z = z_ref[...]
    q = q_ref[...]
    qst = z + (q - z)
    qst_ref[...] = qst
    rec_ref[...] = jnp.dot(qst, W_ref[...],
                           preferred_element_type=jnp.float32) + b_ref[...]


def _recon(z, q, W, b):
    N, CD = z.shape
    M = W.shape[1]
    R2 = 512
    qst, rec = pl.pallas_call(
        _recon_body,
        grid=(N // R2,),
        in_specs=[
            pl.BlockSpec((R2, CD), lambda i: (i, 0)),
            pl.BlockSpec((R2, CD), lambda i: (i, 0)),
            pl.BlockSpec((CD, M), lambda i: (0, 0)),
            pl.BlockSpec((1, M), lambda i: (0, 0)),
        ],
        out_specs=[
            pl.BlockSpec((R2, CD), lambda i: (i, 0)),
            pl.BlockSpec((R2, M), lambda i: (i, 0)),
        ],
        out_shape=[
            jax.ShapeDtypeStruct((N, CD), jnp.float32),
            jax.ShapeDtypeStruct((N, M), jnp.float32),
        ],
        compiler_params=pltpu.CompilerParams(
            dimension_semantics=("arbitrary",)),
    )(z, q, W, b.reshape(1, M))
    return qst, rec


def _sc_gather(emb, idx):
    K, D = emb.shape
    N = idx.shape[0]
    info = plsc.get_sparse_core_info()
    NC, NS = info.num_cores, info.num_subcores
    NW = NC * NS
    b_per_w = N // NW
    nchunk = b_per_w // 128
    idx3 = idx.reshape(NW, nchunk, 128)
    mesh = plsc.VectorSubcoreMesh(core_axis_name="c", subcore_axis_name="s")

    @functools.partial(
        pl.kernel, mesh=mesh,
        out_type=jax.ShapeDtypeStruct((N, D), jnp.float32),
        scratch_types=[
            pltpu.VMEM((nchunk, 128), jnp.int32),
            pltpu.VMEM((128, D), jnp.float32),
            pltpu.SemaphoreType.DMA,
        ],
    )
    def k(table_hbm, idx_hbm, out_hbm, idx_v, rows_v, sem):
        wid = lax.axis_index("s") * NC + lax.axis_index("c")
        pltpu.sync_copy(idx_hbm.at[wid], idx_v)
        for j in range(nchunk):
            pltpu.async_copy(table_hbm.at[idx_v.at[j]], rows_v, sem).wait()
            pltpu.sync_copy(rows_v,
                            out_hbm.at[pl.ds(wid * b_per_w + j * 128, 128)])

    return k(emb, idx3)


def kernel(lm_x, vis_x, lm_W1, lm_b1, lm_g1, lm_be1, lm_W2, lm_b2, lm_g2,
           lm_be2, vis_W1, vis_b1, vis_g1, vis_be1, vis_W2, vis_b2, vis_g2,
           vis_be2, emb, lm_dec_W, lm_dec_b, vis_dec_W, vis_dec_b):
    N = lm_x.shape[0]
    CD = emb.shape[1]

    lm_z, lm_idx, lm_csum = _enc_vq(lm_x, lm_W1, lm_b1, lm_g1, lm_be1,
                                    lm_W2, lm_b2, lm_g2, lm_be2, emb)
    vis_z, vis_idx, vis_csum = _enc_vq(vis_x, vis_W1, vis_b1, vis_g1, vis_be1,
                                       vis_W2, vis_b2, vis_g2, vis_be2, emb)

    lm_qraw = _sc_gather(emb, lm_idx)
    vis_qraw = _sc_gather(emb, vis_idx)

    lm_q, lm_recon = _recon(lm_z, lm_qraw, lm_dec_W, lm_dec_b)
    vis_q, vis_recon = _recon(vis_z, vis_qraw, vis_dec_W, vis_dec_b)

    lm_commit = lm_csum[0, 0] / jnp.float32(N * CD)
    vis_commit = vis_csum[0, 0] / jnp.float32(N * CD)

    return (lm_z, vis_z, lm_q, vis_q, lm_idx, vis_idx,
            lm_commit, vis_commit, lm_recon, vis_recon)


# speedup vs baseline: 1.4586x; 1.2446x over previous
import functools

import jax
import jax.numpy as jnp
from jax import lax
from jax.experimental import pallas as pl
from jax.experimental.pallas import tpu as pltpu
from jax.experimental.pallas import tpu_sc as plsc

EPS = 1e-5

_R = 256
_KB = 1024


def _enc_vq_body(x_ref, W1_ref, b1_ref, g1_ref, be1_ref, W2_ref, b2_ref,
                 g2_ref, be2_ref, emb_ref, se_ref, z_ref, idx_ref, csum_ref):
    x = x_ref[...]
    h = jnp.dot(x, W1_ref[...], preferred_element_type=jnp.float32) + b1_ref[...]
    m = jnp.mean(h, axis=-1, keepdims=True)
    v = jnp.mean((h - m) ** 2, axis=-1, keepdims=True)
    h = (h - m) / jnp.sqrt(v + EPS) * g1_ref[...] + be1_ref[...]
    h = jnp.maximum(h, 0)
    h2 = jnp.dot(h, W2_ref[...], preferred_element_type=jnp.float32) + b2_ref[...]
    m2 = jnp.mean(h2, axis=-1, keepdims=True)
    v2 = jnp.mean((h2 - m2) ** 2, axis=-1, keepdims=True)
    z = (h2 - m2) / jnp.sqrt(v2 + EPS) * g2_ref[...] + be2_ref[...]
    z_ref[...] = z

    sz = jnp.sum(z ** 2, axis=1, keepdims=True)
    K = emb_ref.shape[0]
    z2 = z + z
    vacc = jnp.full((_R, 128), jnp.inf, jnp.float32)
    iacc = jnp.zeros((_R, 128), jnp.int32)
    for kb in range(K // _KB):
        e = emb_ref[pl.ds(kb * _KB, _KB), :]
        dot2 = lax.dot_general(z2, e, (((1,), (1,)), ((), ())),
                               preferred_element_type=jnp.float32)
        se = se_ref[0, pl.ds(kb * _KB, _KB)]
        for c in range(_KB // 128):
            d = (sz - dot2[:, c * 128:(c + 1) * 128]) + se[c * 128:(c + 1) * 128][None, :]
            u = d < vacc
            vacc = jnp.where(u, d, vacc)
            iacc = jnp.where(u, jnp.int32(kb * (_KB // 128) + c), iacc)
    run_v = jnp.min(vacc, axis=1)
    lane = lax.broadcasted_iota(jnp.int32, (_R, 128), 1)
    code = iacc * 128 + lane
    run_i = jnp.min(jnp.where(vacc == run_v[:, None], code, K),
                    axis=1).astype(jnp.int32)
    idx_ref[0, 0, :] = run_i

    @pl.when(pl.program_id(0) == 0)
    def _init():
        csum_ref[...] = jnp.zeros((1, 1), jnp.float32)

    csum_ref[...] += jnp.sum(run_v).reshape(1, 1)


def _se_body(emb_ref, se_ref):
    se_ref[0, :] = jnp.sum(emb_ref[...] ** 2, axis=1)


def _emb_sq(emb):
    K, CD = emb.shape
    RB = 256
    return pl.pallas_call(
        _se_body,
        grid=(K // RB,),
        in_specs=[pl.BlockSpec((RB, CD), lambda i: (i, 0))],
        out_specs=pl.BlockSpec((1, RB), lambda i: (0, i)),
        out_shape=jax.ShapeDtypeStruct((1, K), jnp.float32),
    )(emb)


def _enc_vq(x, W1, b1, g1, be1, W2, b2, g2, be2, emb, se):
    N, IN = x.shape
    H = W1.shape[1]
    CD = W2.shape[1]
    K = emb.shape[0]
    grid = N // _R
    z, idx3, csum = pl.pallas_call(
        _enc_vq_body,
        grid=(grid,),
        in_specs=[
            pl.BlockSpec((_R, IN), lambda i: (i, 0)),
            pl.BlockSpec((IN, H), lambda i: (0, 0)),
            pl.BlockSpec((1, H), lambda i: (0, 0)),
            pl.BlockSpec((1, H), lambda i: (0, 0)),
            pl.BlockSpec((1, H), lambda i: (0, 0)),
            pl.BlockSpec((H, CD), lambda i: (0, 0)),
            pl.BlockSpec((1, CD), lambda i: (0, 0)),
            pl.BlockSpec((1, CD), lambda i: (0, 0)),
            pl.BlockSpec((1, CD), lambda i: (0, 0)),
            pl.BlockSpec((K, CD), lambda i: (0, 0)),
            pl.BlockSpec((1, K), lambda i: (0, 0)),
        ],
        out_specs=[
            pl.BlockSpec((_R, CD), lambda i: (i, 0)),
            pl.BlockSpec((1, 1, _R), lambda i: (i, 0, 0)),
            pl.BlockSpec((1, 1), lambda i: (0, 0)),
        ],
        out_shape=[
            jax.ShapeDtypeStruct((N, CD), jnp.float32),
            jax.ShapeDtypeStruct((grid, 1, _R), jnp.int32),
            jax.ShapeDtypeStruct((1, 1), jnp.float32),
        ],
        compiler_params=pltpu.CompilerParams(
            dimension_semantics=("arbitrary",)),
    )(x, W1, b1.reshape(1, H), g1.reshape(1, H), be1.reshape(1, H),
      W2, b2.reshape(1, CD), g2.reshape(1, CD), be2.reshape(1, CD), emb, se)
    return z, idx3.reshape(N), csum


def _recon_body(z_ref, q_ref, W_ref, b_ref, qst_ref, rec_ref):
    z = z_ref[...]
    q = q_ref[...]
    qst = z + (q - z)
    qst_ref[...] = qst
    rec_ref[...] = jnp.dot(qst, W_ref[...],
                           preferred_element_type=jnp.float32) + b_ref[...]


def _recon(z, q, W, b):
    N, CD = z.shape
    M = W.shape[1]
    R2 = 512
    qst, rec = pl.pallas_call(
        _recon_body,
        grid=(N // R2,),
        in_specs=[
            pl.BlockSpec((R2, CD), lambda i: (i, 0)),
            pl.BlockSpec((R2, CD), lambda i: (i, 0)),
            pl.BlockSpec((CD, M), lambda i: (0, 0)),
            pl.BlockSpec((1, M), lambda i: (0, 0)),
        ],
        out_specs=[
            pl.BlockSpec((R2, CD), lambda i: (i, 0)),
            pl.BlockSpec((R2, M), lambda i: (i, 0)),
        ],
        out_shape=[
            jax.ShapeDtypeStruct((N, CD), jnp.float32),
            jax.ShapeDtypeStruct((N, M), jnp.float32),
        ],
        compiler_params=pltpu.CompilerParams(
            dimension_semantics=("arbitrary",)),
    )(z, q, W, b.reshape(1, M))
    return qst, rec


def _sc_gather(emb, idx):
    K, D = emb.shape
    N = idx.shape[0]
    info = plsc.get_sparse_core_info()
    NC, NS = info.num_cores, info.num_subcores
    NW = NC * NS
    b_per_w = N // NW
    nchunk = b_per_w // 128
    idx3 = idx.reshape(NW, nchunk, 128)
    mesh = plsc.VectorSubcoreMesh(core_axis_name="c", subcore_axis_name="s")

    @functools.partial(
        pl.kernel, mesh=mesh,
        out_type=jax.ShapeDtypeStruct((N, D), jnp.float32),
        scratch_types=[
            pltpu.VMEM((nchunk, 128), jnp.int32),
            pltpu.VMEM((128, D), jnp.float32),
            pltpu.SemaphoreType.DMA,
        ],
    )
    def k(table_hbm, idx_hbm, out_hbm, idx_v, rows_v, sem):
        wid = lax.axis_index("s") * NC + lax.axis_index("c")
        pltpu.sync_copy(idx_hbm.at[wid], idx_v)
        for j in range(nchunk):
            pltpu.async_copy(table_hbm.at[idx_v.at[j]], rows_v, sem).wait()
            pltpu.sync_copy(rows_v,
                            out_hbm.at[pl.ds(wid * b_per_w + j * 128, 128)])

    return k(emb, idx3)


def kernel(lm_x, vis_x, lm_W1, lm_b1, lm_g1, lm_be1, lm_W2, lm_b2, lm_g2,
           lm_be2, vis_W1, vis_b1, vis_g1, vis_be1, vis_W2, vis_b2, vis_g2,
           vis_be2, emb, lm_dec_W, lm_dec_b, vis_dec_W, vis_dec_b):
    N = lm_x.shape[0]
    CD = emb.shape[1]

    se = _emb_sq(emb)
    lm_z, lm_idx, lm_csum = _enc_vq(lm_x, lm_W1, lm_b1, lm_g1, lm_be1,
                                    lm_W2, lm_b2, lm_g2, lm_be2, emb, se)
    vis_z, vis_idx, vis_csum = _enc_vq(vis_x, vis_W1, vis_b1, vis_g1, vis_be1,
                                       vis_W2, vis_b2, vis_g2, vis_be2, emb, se)

    lm_qraw = _sc_gather(emb, lm_idx)
    vis_qraw = _sc_gather(emb, vis_idx)

    lm_q, lm_recon = _recon(lm_z, lm_qraw, lm_dec_W, lm_dec_b)
    vis_q, vis_recon = _recon(vis_z, vis_qraw, vis_dec_W, vis_dec_b)

    lm_commit = lm_csum[0, 0] / jnp.float32(N * CD)
    vis_commit = vis_csum[0, 0] / jnp.float32(N * CD)

    return (lm_z, vis_z, lm_q, vis_q, lm_idx, vis_idx,
            lm_commit, vis_commit, lm_recon, vis_recon)


# speedup vs baseline: 1.5175x; 1.0404x over previous
import functools

import jax
import jax.numpy as jnp
from jax import lax
from jax.experimental import pallas as pl
from jax.experimental.pallas import tpu as pltpu
from jax.experimental.pallas import tpu_sc as plsc

EPS = 1e-5

_R = 256
_KB = 1024


def _encode(x, W1_ref, b1_ref, g1_ref, be1_ref, W2_ref, b2_ref, g2_ref, be2_ref):
    h = jnp.dot(x, W1_ref[...], preferred_element_type=jnp.float32) + b1_ref[...]
    m = jnp.mean(h, axis=-1, keepdims=True)
    v = jnp.mean((h - m) ** 2, axis=-1, keepdims=True)
    h = (h - m) / jnp.sqrt(v + EPS) * g1_ref[...] + be1_ref[...]
    h = jnp.maximum(h, 0)
    h2 = jnp.dot(h, W2_ref[...], preferred_element_type=jnp.float32) + b2_ref[...]
    m2 = jnp.mean(h2, axis=-1, keepdims=True)
    v2 = jnp.mean((h2 - m2) ** 2, axis=-1, keepdims=True)
    return (h2 - m2) / jnp.sqrt(v2 + EPS) * g2_ref[...] + be2_ref[...]


def _vq_scan(z, emb_ref, se_ref, idx_ref, csum_ref):
    K = emb_ref.shape[0]
    sz = jnp.sum(z ** 2, axis=1, keepdims=True)
    z2 = z + z
    vacc = jnp.full((_R, 128), jnp.inf, jnp.float32)
    iacc = jnp.zeros((_R, 128), jnp.int32)
    for kb in range(K // _KB):
        e = emb_ref[pl.ds(kb * _KB, _KB), :]
        dot2 = lax.dot_general(z2, e, (((1,), (1,)), ((), ())),
                               preferred_element_type=jnp.float32)
        se = se_ref[0, pl.ds(kb * _KB, _KB)]
        for c in range(_KB // 128):
            d = (sz - dot2[:, c * 128:(c + 1) * 128]) + se[c * 128:(c + 1) * 128][None, :]
            u = d < vacc
            vacc = jnp.where(u, d, vacc)
            iacc = jnp.where(u, jnp.int32(kb * (_KB // 128) + c), iacc)
    run_v = jnp.min(vacc, axis=1)
    lane = lax.broadcasted_iota(jnp.int32, (_R, 128), 1)
    code = iacc * 128 + lane
    run_i = jnp.min(jnp.where(vacc == run_v[:, None], code, K),
                    axis=1).astype(jnp.int32)
    idx_ref[0, 0, :] = run_i

    @pl.when(pl.program_id(0) == 0)
    def _init():
        csum_ref[...] = jnp.zeros((1, 1), jnp.float32)

    csum_ref[...] += jnp.sum(run_v).reshape(1, 1)


def _enc_vq_body(lx_ref, lW1_ref, lb1_ref, lg1_ref, lbe1_ref, lW2_ref, lb2_ref,
                 lg2_ref, lbe2_ref,
                 vx_ref, vW1_ref, vb1_ref, vg1_ref, vbe1_ref, vW2_ref, vb2_ref,
                 vg2_ref, vbe2_ref, emb_ref,
                 lz_ref, lidx_ref, lcsum_ref, vz_ref, vidx_ref, vcsum_ref,
                 se_ref):
    @pl.when(pl.program_id(0) == 0)
    def _se():
        se_ref[0, :] = jnp.sum(emb_ref[...] ** 2, axis=1)

    lz = _encode(lx_ref[...], lW1_ref, lb1_ref, lg1_ref, lbe1_ref,
                 lW2_ref, lb2_ref, lg2_ref, lbe2_ref)
    lz_ref[...] = lz
    _vq_scan(lz, emb_ref, se_ref, lidx_ref, lcsum_ref)

    vz = _encode(vx_ref[...], vW1_ref, vb1_ref, vg1_ref, vbe1_ref,
                 vW2_ref, vb2_ref, vg2_ref, vbe2_ref)
    vz_ref[...] = vz
    _vq_scan(vz, emb_ref, se_ref, vidx_ref, vcsum_ref)


def _enc_vq_both(lm_x, lm_w, vis_x, vis_w, emb):
    N = lm_x.shape[0]
    LM = lm_x.shape[1]
    VIS = vis_x.shape[1]
    H = lm_w[0].shape[1]
    CD = emb.shape[1]
    K = emb.shape[0]
    grid = N // _R

    def vec(b):
        return b.reshape(1, -1)

    full = lambda i: (0, 0)
    row = lambda i: (i, 0)
    specs = ([pl.BlockSpec((_R, LM), row),
              pl.BlockSpec((LM, H), full)] +
             [pl.BlockSpec((1, H), full)] * 3 +
             [pl.BlockSpec((H, CD), full)] +
             [pl.BlockSpec((1, CD), full)] * 3 +
             [pl.BlockSpec((_R, VIS), row),
              pl.BlockSpec((VIS, H), full)] +
             [pl.BlockSpec((1, H), full)] * 3 +
             [pl.BlockSpec((H, CD), full)] +
             [pl.BlockSpec((1, CD), full)] * 3 +
             [pl.BlockSpec((K, CD), full)])
    out_specs = [
        pl.BlockSpec((_R, CD), row),
        pl.BlockSpec((1, 1, _R), lambda i: (i, 0, 0)),
        pl.BlockSpec((1, 1), full),
        pl.BlockSpec((_R, CD), row),
        pl.BlockSpec((1, 1, _R), lambda i: (i, 0, 0)),
        pl.BlockSpec((1, 1), full),
    ]
    out_shape = [
        jax.ShapeDtypeStruct((N, CD), jnp.float32),
        jax.ShapeDtypeStruct((grid, 1, _R), jnp.int32),
        jax.ShapeDtypeStruct((1, 1), jnp.float32),
        jax.ShapeDtypeStruct((N, CD), jnp.float32),
        jax.ShapeDtypeStruct((grid, 1, _R), jnp.int32),
        jax.ShapeDtypeStruct((1, 1), jnp.float32),
    ]
    lW1, lb1, lg1, lbe1, lW2, lb2, lg2, lbe2 = lm_w
    vW1, vb1, vg1, vbe1, vW2, vb2, vg2, vbe2 = vis_w
    lz, lidx, lcsum, vz, vidx, vcsum = pl.pallas_call(
        _enc_vq_body,
        grid=(grid,),
        in_specs=specs,
        out_specs=out_specs,
        out_shape=out_shape,
        scratch_shapes=[pltpu.VMEM((1, K), jnp.float32)],
        compiler_params=pltpu.CompilerParams(
            dimension_semantics=("arbitrary",)),
    )(lm_x, lW1, vec(lb1), vec(lg1), vec(lbe1), lW2, vec(lb2), vec(lg2), vec(lbe2),
      vis_x, vW1, vec(vb1), vec(vg1), vec(vbe1), vW2, vec(vb2), vec(vg2), vec(vbe2),
      emb)
    return lz, lidx.reshape(N), lcsum, vz, vidx.reshape(N), vcsum


def _recon_body(lz_ref, lq_ref, lW_ref, lb_ref, vz_ref, vq_ref, vW_ref, vb_ref,
                lqst_ref, lrec_ref, vqst_ref, vrec_ref):
    lz = lz_ref[...]
    lq = lq_ref[...]
    lqst = lz + (lq - lz)
    lqst_ref[...] = lqst
    lrec_ref[...] = jnp.dot(lqst, lW_ref[...],
                            preferred_element_type=jnp.float32) + lb_ref[...]
    vz = vz_ref[...]
    vq = vq_ref[...]
    vqst = vz + (vq - vz)
    vqst_ref[...] = vqst
    vrec_ref[...] = jnp.dot(vqst, vW_ref[...],
                            preferred_element_type=jnp.float32) + vb_ref[...]


def _recon_both(lz, lq, lW, lb, vz, vq, vW, vb):
    N, CD = lz.shape
    ML = lW.shape[1]
    MV = vW.shape[1]
    R2 = 512
    full = lambda i: (0, 0)
    row = lambda i: (i, 0)
    lqst, lrec, vqst, vrec = pl.pallas_call(
        _recon_body,
        grid=(N // R2,),
        in_specs=[
            pl.BlockSpec((R2, CD), row),
            pl.BlockSpec((R2, CD), row),
            pl.BlockSpec((CD, ML), full),
            pl.BlockSpec((1, ML), full),
            pl.BlockSpec((R2, CD), row),
            pl.BlockSpec((R2, CD), row),
            pl.BlockSpec((CD, MV), full),
            pl.BlockSpec((1, MV), full),
        ],
        out_specs=[
            pl.BlockSpec((R2, CD), row),
            pl.BlockSpec((R2, ML), row),
            pl.BlockSpec((R2, CD), row),
            pl.BlockSpec((R2, MV), row),
        ],
        out_shape=[
            jax.ShapeDtypeStruct((N, CD), jnp.float32),
            jax.ShapeDtypeStruct((N, ML), jnp.float32),
            jax.ShapeDtypeStruct((N, CD), jnp.float32),
            jax.ShapeDtypeStruct((N, MV), jnp.float32),
        ],
        compiler_params=pltpu.CompilerParams(
            dimension_semantics=("arbitrary",)),
    )(lz, lq, lW, lb.reshape(1, ML), vz, vq, vW, vb.reshape(1, MV))
    return lqst, lrec, vqst, vrec


def _sc_gather_both(emb, lidx, vidx):
    K, D = emb.shape
    N = lidx.shape[0]
    info = plsc.get_sparse_core_info()
    NC, NS = info.num_cores, info.num_subcores
    NW = NC * NS
    b_per_w = N // NW
    nchunk = b_per_w // 128
    li3 = lidx.reshape(NW, nchunk, 128)
    vi3 = vidx.reshape(NW, nchunk, 128)
    mesh = plsc.VectorSubcoreMesh(core_axis_name="c", subcore_axis_name="s")

    @functools.partial(
        pl.kernel, mesh=mesh,
        out_type=[jax.ShapeDtypeStruct((N, D), jnp.float32),
                  jax.ShapeDtypeStruct((N, D), jnp.float32)],
        scratch_types=[
            pltpu.VMEM((nchunk, 128), jnp.int32),
            pltpu.VMEM((nchunk, 128), jnp.int32),
            pltpu.VMEM((128, D), jnp.float32),
            pltpu.VMEM((128, D), jnp.float32),
            pltpu.SemaphoreType.DMA,
            pltpu.SemaphoreType.DMA,
        ],
    )
    def k(table_hbm, li_hbm, vi_hbm, lout_hbm, vout_hbm,
          li_v, vi_v, rows0, rows1, sem0, sem1):
        wid = lax.axis_index("s") * NC + lax.axis_index("c")
        base = wid * b_per_w
        pltpu.sync_copy(li_hbm.at[wid], li_v)
        pltpu.sync_copy(vi_hbm.at[wid], vi_v)
        for idx_v, out_hbm in ((li_v, lout_hbm), (vi_v, vout_hbm)):
            cps = [pltpu.async_copy(table_hbm.at[idx_v.at[j]],
                                    (rows0, rows1)[j], (sem0, sem1)[j])
                   for j in range(nchunk)]
            for j in range(nchunk):
                cps[j].wait()
                pltpu.sync_copy((rows0, rows1)[j],
                                out_hbm.at[pl.ds(base + j * 128, 128)])

    return k(emb, li3, vi3)


def kernel(lm_x, vis_x, lm_W1, lm_b1, lm_g1, lm_be1, lm_W2, lm_b2, lm_g2,
           lm_be2, vis_W1, vis_b1, vis_g1, vis_be1, vis_W2, vis_b2, vis_g2,
           vis_be2, emb, lm_dec_W, lm_dec_b, vis_dec_W, vis_dec_b):
    N = lm_x.shape[0]
    CD = emb.shape[1]

    lm_z, lm_idx, lm_csum, vis_z, vis_idx, vis_csum = _enc_vq_both(
        lm_x, (lm_W1, lm_b1, lm_g1, lm_be1, lm_W2, lm_b2, lm_g2, lm_be2),
        vis_x, (vis_W1, vis_b1, vis_g1, vis_be1, vis_W2, vis_b2, vis_g2, vis_be2),
        emb)

    lm_qraw, vis_qraw = _sc_gather_both(emb, lm_idx, vis_idx)

    lm_q, lm_recon, vis_q, vis_recon = _recon_both(
        lm_z, lm_qraw, lm_dec_W, lm_dec_b, vis_z, vis_qraw, vis_dec_W, vis_dec_b)

    lm_commit = lm_csum[0, 0] / jnp.float32(N * CD)
    vis_commit = vis_csum[0, 0] / jnp.float32(N * CD)

    return (lm_z, vis_z, lm_q, vis_q, lm_idx, vis_idx,
            lm_commit, vis_commit, lm_recon, vis_recon)


# speedup vs baseline: 1.7002x; 1.1204x over previous
import functools

import jax
import jax.numpy as jnp
from jax import lax
from jax.experimental import pallas as pl
from jax.experimental.pallas import tpu as pltpu
from jax.experimental.pallas import tpu_sc as plsc

EPS = 1e-5

_R = 512
_KB = 2048


def _encode(x, W1_ref, b1_ref, g1_ref, be1_ref, W2_ref, b2_ref, g2_ref, be2_ref):
    h = jnp.dot(x, W1_ref[...], preferred_element_type=jnp.float32) + b1_ref[...]
    m = jnp.mean(h, axis=-1, keepdims=True)
    v = jnp.mean((h - m) ** 2, axis=-1, keepdims=True)
    h = (h - m) / jnp.sqrt(v + EPS) * g1_ref[...] + be1_ref[...]
    h = jnp.maximum(h, 0)
    h2 = jnp.dot(h, W2_ref[...], preferred_element_type=jnp.float32) + b2_ref[...]
    m2 = jnp.mean(h2, axis=-1, keepdims=True)
    v2 = jnp.mean((h2 - m2) ** 2, axis=-1, keepdims=True)
    return (h2 - m2) / jnp.sqrt(v2 + EPS) * g2_ref[...] + be2_ref[...]


def _vq_scan(z, emb_ref, se_ref, idx_ref, csum_ref):
    K = emb_ref.shape[0]
    sz = jnp.sum(z ** 2, axis=1, keepdims=True)
    z2 = z + z
    vacc = jnp.full((_R, 128), jnp.inf, jnp.float32)
    iacc = jnp.zeros((_R, 128), jnp.int32)
    for kb in range(K // _KB):
        e = emb_ref[pl.ds(kb * _KB, _KB), :]
        dot2 = lax.dot_general(z2, e, (((1,), (1,)), ((), ())),
                               preferred_element_type=jnp.float32)
        se = se_ref[0, pl.ds(kb * _KB, _KB)]
        for c in range(_KB // 128):
            d = (sz - dot2[:, c * 128:(c + 1) * 128]) + se[c * 128:(c + 1) * 128][None, :]
            u = d < vacc
            vacc = jnp.where(u, d, vacc)
            iacc = jnp.where(u, jnp.int32(kb * (_KB // 128) + c), iacc)
    run_v = jnp.min(vacc, axis=1)
    lane = lax.broadcasted_iota(jnp.int32, (_R, 128), 1)
    code = iacc * 128 + lane
    run_i = jnp.min(jnp.where(vacc == run_v[:, None], code, K),
                    axis=1).astype(jnp.int32)
    idx_ref[0, 0, :] = run_i

    @pl.when(pl.program_id(0) == 0)
    def _init():
        csum_ref[...] = jnp.zeros((1, 1), jnp.float32)

    csum_ref[...] += jnp.sum(run_v).reshape(1, 1)


def _enc_vq_body(lx_ref, lW1_ref, lb1_ref, lg1_ref, lbe1_ref, lW2_ref, lb2_ref,
                 lg2_ref, lbe2_ref,
                 vx_ref, vW1_ref, vb1_ref, vg1_ref, vbe1_ref, vW2_ref, vb2_ref,
                 vg2_ref, vbe2_ref, emb_ref,
                 lz_ref, lidx_ref, lcsum_ref, vz_ref, vidx_ref, vcsum_ref,
                 se_ref):
    @pl.when(pl.program_id(0) == 0)
    def _se():
        se_ref[0, :] = jnp.sum(emb_ref[...] ** 2, axis=1)

    lz = _encode(lx_ref[...], lW1_ref, lb1_ref, lg1_ref, lbe1_ref,
                 lW2_ref, lb2_ref, lg2_ref, lbe2_ref)
    lz_ref[...] = lz
    _vq_scan(lz, emb_ref, se_ref, lidx_ref, lcsum_ref)

    vz = _encode(vx_ref[...], vW1_ref, vb1_ref, vg1_ref, vbe1_ref,
                 vW2_ref, vb2_ref, vg2_ref, vbe2_ref)
    vz_ref[...] = vz
    _vq_scan(vz, emb_ref, se_ref, vidx_ref, vcsum_ref)


def _enc_vq_both(lm_x, lm_w, vis_x, vis_w, emb):
    N = lm_x.shape[0]
    LM = lm_x.shape[1]
    VIS = vis_x.shape[1]
    H = lm_w[0].shape[1]
    CD = emb.shape[1]
    K = emb.shape[0]
    grid = N // _R

    def vec(b):
        return b.reshape(1, -1)

    full = lambda i: (0, 0)
    row = lambda i: (i, 0)
    specs = ([pl.BlockSpec((_R, LM), row),
              pl.BlockSpec((LM, H), full)] +
             [pl.BlockSpec((1, H), full)] * 3 +
             [pl.BlockSpec((H, CD), full)] +
             [pl.BlockSpec((1, CD), full)] * 3 +
             [pl.BlockSpec((_R, VIS), row),
              pl.BlockSpec((VIS, H), full)] +
             [pl.BlockSpec((1, H), full)] * 3 +
             [pl.BlockSpec((H, CD), full)] +
             [pl.BlockSpec((1, CD), full)] * 3 +
             [pl.BlockSpec((K, CD), full)])
    out_specs = [
        pl.BlockSpec((_R, CD), row),
        pl.BlockSpec((1, 1, _R), lambda i: (i, 0, 0)),
        pl.BlockSpec((1, 1), full),
        pl.BlockSpec((_R, CD), row),
        pl.BlockSpec((1, 1, _R), lambda i: (i, 0, 0)),
        pl.BlockSpec((1, 1), full),
    ]
    out_shape = [
        jax.ShapeDtypeStruct((N, CD), jnp.float32),
        jax.ShapeDtypeStruct((grid, 1, _R), jnp.int32),
        jax.ShapeDtypeStruct((1, 1), jnp.float32),
        jax.ShapeDtypeStruct((N, CD), jnp.float32),
        jax.ShapeDtypeStruct((grid, 1, _R), jnp.int32),
        jax.ShapeDtypeStruct((1, 1), jnp.float32),
    ]
    lW1, lb1, lg1, lbe1, lW2, lb2, lg2, lbe2 = lm_w
    vW1, vb1, vg1, vbe1, vW2, vb2, vg2, vbe2 = vis_w
    lz, lidx, lcsum, vz, vidx, vcsum = pl.pallas_call(
        _enc_vq_body,
        grid=(grid,),
        in_specs=specs,
        out_specs=out_specs,
        out_shape=out_shape,
        scratch_shapes=[pltpu.VMEM((1, K), jnp.float32)],
        compiler_params=pltpu.CompilerParams(
            dimension_semantics=("arbitrary",)),
    )(lm_x, lW1, vec(lb1), vec(lg1), vec(lbe1), lW2, vec(lb2), vec(lg2), vec(lbe2),
      vis_x, vW1, vec(vb1), vec(vg1), vec(vbe1), vW2, vec(vb2), vec(vg2), vec(vbe2),
      emb)
    return lz, lidx.reshape(N), lcsum, vz, vidx.reshape(N), vcsum


def _recon_body(lz_ref, lq_ref, lW_ref, lb_ref, vz_ref, vq_ref, vW_ref, vb_ref,
                lqst_ref, lrec_ref, vqst_ref, vrec_ref):
    lz = lz_ref[...]
    lq = lq_ref[...]
    lqst = lz + (lq - lz)
    lqst_ref[...] = lqst
    lrec_ref[...] = jnp.dot(lqst, lW_ref[...],
                            preferred_element_type=jnp.float32) + lb_ref[...]
    vz = vz_ref[...]
    vq = vq_ref[...]
    vqst = vz + (vq - vz)
    vqst_ref[...] = vqst
    vrec_ref[...] = jnp.dot(vqst, vW_ref[...],
                            preferred_element_type=jnp.float32) + vb_ref[...]


def _recon_both(lz, lq, lW, lb, vz, vq, vW, vb):
    N, CD = lz.shape
    ML = lW.shape[1]
    MV = vW.shape[1]
    R2 = 1024
    full = lambda i: (0, 0)
    row = lambda i: (i, 0)
    lqst, lrec, vqst, vrec = pl.pallas_call(
        _recon_body,
        grid=(N // R2,),
        in_specs=[
            pl.BlockSpec((R2, CD), row),
            pl.BlockSpec((R2, CD), row),
            pl.BlockSpec((CD, ML), full),
            pl.BlockSpec((1, ML), full),
            pl.BlockSpec((R2, CD), row),
            pl.BlockSpec((R2, CD), row),
            pl.BlockSpec((CD, MV), full),
            pl.BlockSpec((1, MV), full),
        ],
        out_specs=[
            pl.BlockSpec((R2, CD), row),
            pl.BlockSpec((R2, ML), row),
            pl.BlockSpec((R2, CD), row),
            pl.BlockSpec((R2, MV), row),
        ],
        out_shape=[
            jax.ShapeDtypeStruct((N, CD), jnp.float32),
            jax.ShapeDtypeStruct((N, ML), jnp.float32),
            jax.ShapeDtypeStruct((N, CD), jnp.float32),
            jax.ShapeDtypeStruct((N, MV), jnp.float32),
        ],
        compiler_params=pltpu.CompilerParams(
            dimension_semantics=("arbitrary",)),
    )(lz, lq, lW, lb.reshape(1, ML), vz, vq, vW, vb.reshape(1, MV))
    return lqst, lrec, vqst, vrec


def _sc_gather_both(emb, lidx, vidx):
    K, D = emb.shape
    N = lidx.shape[0]
    info = plsc.get_sparse_core_info()
    NC, NS = info.num_cores, info.num_subcores
    NW = NC * NS
    b_per_w = N // NW
    nchunk = b_per_w // 128
    li3 = lidx.reshape(NW, nchunk, 128)
    vi3 = vidx.reshape(NW, nchunk, 128)
    mesh = plsc.VectorSubcoreMesh(core_axis_name="c", subcore_axis_name="s")

    @functools.partial(
        pl.kernel, mesh=mesh,
        out_type=[jax.ShapeDtypeStruct((N, D), jnp.float32),
                  jax.ShapeDtypeStruct((N, D), jnp.float32)],
        scratch_types=[
            pltpu.VMEM((nchunk, 128), jnp.int32),
            pltpu.VMEM((nchunk, 128), jnp.int32),
            pltpu.VMEM((128, D), jnp.float32),
            pltpu.VMEM((128, D), jnp.float32),
            pltpu.SemaphoreType.DMA,
            pltpu.SemaphoreType.DMA,
        ],
    )
    def k(table_hbm, li_hbm, vi_hbm, lout_hbm, vout_hbm,
          li_v, vi_v, rows0, rows1, sem0, sem1):
        wid = lax.axis_index("s") * NC + lax.axis_index("c")
        base = wid * b_per_w
        pltpu.sync_copy(li_hbm.at[wid], li_v)
        pltpu.sync_copy(vi_hbm.at[wid], vi_v)
        for idx_v, out_hbm in ((li_v, lout_hbm), (vi_v, vout_hbm)):
            cps = [pltpu.async_copy(table_hbm.at[idx_v.at[j]],
                                    (rows0, rows1)[j], (sem0, sem1)[j])
                   for j in range(nchunk)]
            for j in range(nchunk):
                cps[j].wait()
                pltpu.sync_copy((rows0, rows1)[j],
                                out_hbm.at[pl.ds(base + j * 128, 128)])

    return k(emb, li3, vi3)


def kernel(lm_x, vis_x, lm_W1, lm_b1, lm_g1, lm_be1, lm_W2, lm_b2, lm_g2,
           lm_be2, vis_W1, vis_b1, vis_g1, vis_be1, vis_W2, vis_b2, vis_g2,
           vis_be2, emb, lm_dec_W, lm_dec_b, vis_dec_W, vis_dec_b):
    N = lm_x.shape[0]
    CD = emb.shape[1]

    lm_z, lm_idx, lm_csum, vis_z, vis_idx, vis_csum = _enc_vq_both(
        lm_x, (lm_W1, lm_b1, lm_g1, lm_be1, lm_W2, lm_b2, lm_g2, lm_be2),
        vis_x, (vis_W1, vis_b1, vis_g1, vis_be1, vis_W2, vis_b2, vis_g2, vis_be2),
        emb)

    lm_qraw, vis_qraw = _sc_gather_both(emb, lm_idx, vis_idx)

    lm_q, lm_recon, vis_q, vis_recon = _recon_both(
        lm_z, lm_qraw, lm_dec_W, lm_dec_b, vis_z, vis_qraw, vis_dec_W, vis_dec_b)

    lm_commit = lm_csum[0, 0] / jnp.float32(N * CD)
    vis_commit = vis_csum[0, 0] / jnp.float32(N * CD)

    return (lm_z, vis_z, lm_q, vis_q, lm_idx, vis_idx,
            lm_commit, vis_commit, lm_recon, vis_recon)


# speedup vs baseline: 1.7862x; 1.0506x over previous
import functools

import jax
import jax.numpy as jnp
from jax import lax
from jax.experimental import pallas as pl
from jax.experimental.pallas import tpu as pltpu
from jax.experimental.pallas import tpu_sc as plsc

EPS = 1e-5

_R = 1024
_KB = 2048


def _encode(x, W1_ref, b1_ref, g1_ref, be1_ref, W2_ref, b2_ref, g2_ref, be2_ref):
    h = jnp.dot(x, W1_ref[...], preferred_element_type=jnp.float32) + b1_ref[...]
    m = jnp.mean(h, axis=-1, keepdims=True)
    v = jnp.mean((h - m) ** 2, axis=-1, keepdims=True)
    h = (h - m) / jnp.sqrt(v + EPS) * g1_ref[...] + be1_ref[...]
    h = jnp.maximum(h, 0)
    h2 = jnp.dot(h, W2_ref[...], preferred_element_type=jnp.float32) + b2_ref[...]
    m2 = jnp.mean(h2, axis=-1, keepdims=True)
    v2 = jnp.mean((h2 - m2) ** 2, axis=-1, keepdims=True)
    return (h2 - m2) / jnp.sqrt(v2 + EPS) * g2_ref[...] + be2_ref[...]


def _vq_scan(z, emb_ref, se_ref, idx_ref, csum_ref):
    K = emb_ref.shape[0]
    sz = jnp.sum(z ** 2, axis=1, keepdims=True)
    z2 = z + z
    vacc = jnp.full((_R, 128), jnp.inf, jnp.float32)
    iacc = jnp.zeros((_R, 128), jnp.int32)
    for kb in range(K // _KB):
        e = emb_ref[pl.ds(kb * _KB, _KB), :]
        dot2 = lax.dot_general(z2, e, (((1,), (1,)), ((), ())),
                               preferred_element_type=jnp.float32)
        se = se_ref[0, pl.ds(kb * _KB, _KB)]
        for c in range(_KB // 128):
            d = (sz - dot2[:, c * 128:(c + 1) * 128]) + se[c * 128:(c + 1) * 128][None, :]
            u = d < vacc
            vacc = jnp.where(u, d, vacc)
            iacc = jnp.where(u, jnp.int32(kb * (_KB // 128) + c), iacc)
    run_v = jnp.min(vacc, axis=1)
    lane = lax.broadcasted_iota(jnp.int32, (_R, 128), 1)
    code = iacc * 128 + lane
    run_i = jnp.min(jnp.where(vacc == run_v[:, None], code, K),
                    axis=1).astype(jnp.int32)
    idx_ref[0, 0, :] = run_i

    @pl.when(pl.program_id(0) == 0)
    def _init():
        csum_ref[...] = jnp.zeros((1, 1), jnp.float32)

    csum_ref[...] += jnp.sum(run_v).reshape(1, 1)


def _enc_vq_body(lx_ref, lW1_ref, lb1_ref, lg1_ref, lbe1_ref, lW2_ref, lb2_ref,
                 lg2_ref, lbe2_ref,
                 vx_ref, vW1_ref, vb1_ref, vg1_ref, vbe1_ref, vW2_ref, vb2_ref,
                 vg2_ref, vbe2_ref, emb_ref,
                 lz_ref, lidx_ref, lcsum_ref, vz_ref, vidx_ref, vcsum_ref,
                 se_ref):
    @pl.when(pl.program_id(0) == 0)
    def _se():
        se_ref[0, :] = jnp.sum(emb_ref[...] ** 2, axis=1)

    lz = _encode(lx_ref[...], lW1_ref, lb1_ref, lg1_ref, lbe1_ref,
                 lW2_ref, lb2_ref, lg2_ref, lbe2_ref)
    lz_ref[...] = lz
    _vq_scan(lz, emb_ref, se_ref, lidx_ref, lcsum_ref)

    vz = _encode(vx_ref[...], vW1_ref, vb1_ref, vg1_ref, vbe1_ref,
                 vW2_ref, vb2_ref, vg2_ref, vbe2_ref)
    vz_ref[...] = vz
    _vq_scan(vz, emb_ref, se_ref, vidx_ref, vcsum_ref)


def _enc_vq_both(lm_x, lm_w, vis_x, vis_w, emb):
    N = lm_x.shape[0]
    LM = lm_x.shape[1]
    VIS = vis_x.shape[1]
    H = lm_w[0].shape[1]
    CD = emb.shape[1]
    K = emb.shape[0]
    grid = N // _R

    def vec(b):
        return b.reshape(1, -1)

    full = lambda i: (0, 0)
    row = lambda i: (i, 0)
    specs = ([pl.BlockSpec((_R, LM), row),
              pl.BlockSpec((LM, H), full)] +
             [pl.BlockSpec((1, H), full)] * 3 +
             [pl.BlockSpec((H, CD), full)] +
             [pl.BlockSpec((1, CD), full)] * 3 +
             [pl.BlockSpec((_R, VIS), row),
              pl.BlockSpec((VIS, H), full)] +
             [pl.BlockSpec((1, H), full)] * 3 +
             [pl.BlockSpec((H, CD), full)] +
             [pl.BlockSpec((1, CD), full)] * 3 +
             [pl.BlockSpec((K, CD), full)])
    out_specs = [
        pl.BlockSpec((_R, CD), row),
        pl.BlockSpec((1, 1, _R), lambda i: (i, 0, 0)),
        pl.BlockSpec((1, 1), full),
        pl.BlockSpec((_R, CD), row),
        pl.BlockSpec((1, 1, _R), lambda i: (i, 0, 0)),
        pl.BlockSpec((1, 1), full),
    ]
    out_shape = [
        jax.ShapeDtypeStruct((N, CD), jnp.float32),
        jax.ShapeDtypeStruct((grid, 1, _R), jnp.int32),
        jax.ShapeDtypeStruct((1, 1), jnp.float32),
        jax.ShapeDtypeStruct((N, CD), jnp.float32),
        jax.ShapeDtypeStruct((grid, 1, _R), jnp.int32),
        jax.ShapeDtypeStruct((1, 1), jnp.float32),
    ]
    lW1, lb1, lg1, lbe1, lW2, lb2, lg2, lbe2 = lm_w
    vW1, vb1, vg1, vbe1, vW2, vb2, vg2, vbe2 = vis_w
    lz, lidx, lcsum, vz, vidx, vcsum = pl.pallas_call(
        _enc_vq_body,
        grid=(grid,),
        in_specs=specs,
        out_specs=out_specs,
        out_shape=out_shape,
        scratch_shapes=[pltpu.VMEM((1, K), jnp.float32)],
        compiler_params=pltpu.CompilerParams(
            dimension_semantics=("arbitrary",)),
    )(lm_x, lW1, vec(lb1), vec(lg1), vec(lbe1), lW2, vec(lb2), vec(lg2), vec(lbe2),
      vis_x, vW1, vec(vb1), vec(vg1), vec(vbe1), vW2, vec(vb2), vec(vg2), vec(vbe2),
      emb)
    return lz, lidx.reshape(N), lcsum, vz, vidx.reshape(N), vcsum


def _recon_body(lz_ref, lq_ref, lW_ref, lb_ref, vz_ref, vq_ref, vW_ref, vb_ref,
                lqst_ref, lrec_ref, vqst_ref, vrec_ref):
    lz = lz_ref[...]
    lq = lq_ref[...]
    lqst = lz + (lq - lz)
    lqst_ref[...] = lqst
    lrec_ref[...] = jnp.dot(lqst, lW_ref[...],
                            preferred_element_type=jnp.float32) + lb_ref[...]
    vz = vz_ref[...]
    vq = vq_ref[...]
    vqst = vz + (vq - vz)
    vqst_ref[...] = vqst
    vrec_ref[...] = jnp.dot(vqst, vW_ref[...],
                            preferred_element_type=jnp.float32) + vb_ref[...]


def _recon_both(lz, lq, lW, lb, vz, vq, vW, vb):
    N, CD = lz.shape
    ML = lW.shape[1]
    MV = vW.shape[1]
    R2 = 2048
    full = lambda i: (0, 0)
    row = lambda i: (i, 0)
    lqst, lrec, vqst, vrec = pl.pallas_call(
        _recon_body,
        grid=(N // R2,),
        in_specs=[
            pl.BlockSpec((R2, CD), row),
            pl.BlockSpec((R2, CD), row),
            pl.BlockSpec((CD, ML), full),
            pl.BlockSpec((1, ML), full),
            pl.BlockSpec((R2, CD), row),
            pl.BlockSpec((R2, CD), row),
            pl.BlockSpec((CD, MV), full),
            pl.BlockSpec((1, MV), full),
        ],
        out_specs=[
            pl.BlockSpec((R2, CD), row),
            pl.BlockSpec((R2, ML), row),
            pl.BlockSpec((R2, CD), row),
            pl.BlockSpec((R2, MV), row),
        ],
        out_shape=[
            jax.ShapeDtypeStruct((N, CD), jnp.float32),
            jax.ShapeDtypeStruct((N, ML), jnp.float32),
            jax.ShapeDtypeStruct((N, CD), jnp.float32),
            jax.ShapeDtypeStruct((N, MV), jnp.float32),
        ],
        compiler_params=pltpu.CompilerParams(
            dimension_semantics=("arbitrary",)),
    )(lz, lq, lW, lb.reshape(1, ML), vz, vq, vW, vb.reshape(1, MV))
    return lqst, lrec, vqst, vrec


def _sc_gather_both(emb, lidx, vidx):
    K, D = emb.shape
    N = lidx.shape[0]
    info = plsc.get_sparse_core_info()
    NC, NS = info.num_cores, info.num_subcores
    NW = NC * NS
    b_per_w = N // NW
    nchunk = b_per_w // 128
    li3 = lidx.reshape(NW, nchunk, 128)
    vi3 = vidx.reshape(NW, nchunk, 128)
    mesh = plsc.VectorSubcoreMesh(core_axis_name="c", subcore_axis_name="s")

    @functools.partial(
        pl.kernel, mesh=mesh,
        out_type=[jax.ShapeDtypeStruct((N, D), jnp.float32),
                  jax.ShapeDtypeStruct((N, D), jnp.float32)],
        scratch_types=[
            pltpu.VMEM((nchunk, 128), jnp.int32),
            pltpu.VMEM((nchunk, 128), jnp.int32),
            pltpu.VMEM((128, D), jnp.float32),
            pltpu.VMEM((128, D), jnp.float32),
            pltpu.SemaphoreType.DMA,
            pltpu.SemaphoreType.DMA,
        ],
    )
    def k(table_hbm, li_hbm, vi_hbm, lout_hbm, vout_hbm,
          li_v, vi_v, rows0, rows1, sem0, sem1):
        wid = lax.axis_index("s") * NC + lax.axis_index("c")
        base = wid * b_per_w
        pltpu.sync_copy(li_hbm.at[wid], li_v)
        pltpu.sync_copy(vi_hbm.at[wid], vi_v)
        for idx_v, out_hbm in ((li_v, lout_hbm), (vi_v, vout_hbm)):
            cps = [pltpu.async_copy(table_hbm.at[idx_v.at[j]],
                                    (rows0, rows1)[j], (sem0, sem1)[j])
                   for j in range(nchunk)]
            for j in range(nchunk):
                cps[j].wait()
                pltpu.sync_copy((rows0, rows1)[j],
                                out_hbm.at[pl.ds(base + j * 128, 128)])

    return k(emb, li3, vi3)


def kernel(lm_x, vis_x, lm_W1, lm_b1, lm_g1, lm_be1, lm_W2, lm_b2, lm_g2,
           lm_be2, vis_W1, vis_b1, vis_g1, vis_be1, vis_W2, vis_b2, vis_g2,
           vis_be2, emb, lm_dec_W, lm_dec_b, vis_dec_W, vis_dec_b):
    N = lm_x.shape[0]
    CD = emb.shape[1]

    lm_z, lm_idx, lm_csum, vis_z, vis_idx, vis_csum = _enc_vq_both(
        lm_x, (lm_W1, lm_b1, lm_g1, lm_be1, lm_W2, lm_b2, lm_g2, lm_be2),
        vis_x, (vis_W1, vis_b1, vis_g1, vis_be1, vis_W2, vis_b2, vis_g2, vis_be2),
        emb)

    lm_qraw, vis_qraw = _sc_gather_both(emb, lm_idx, vis_idx)

    lm_q, lm_recon, vis_q, vis_recon = _recon_both(
        lm_z, lm_qraw, lm_dec_W, lm_dec_b, vis_z, vis_qraw, vis_dec_W, vis_dec_b)

    lm_commit = lm_csum[0, 0] / jnp.float32(N * CD)
    vis_commit = vis_csum[0, 0] / jnp.float32(N * CD)

    return (lm_z, vis_z, lm_q, vis_q, lm_idx, vis_idx,
            lm_commit, vis_commit, lm_recon, vis_recon)
